# baseline (speedup 1.0000x reference)
import functools

import jax
import jax.numpy as jnp
from jax.experimental import pallas as pl
from jax.experimental.pallas import tpu as pltpu

_MIB = 1024 * 1024
_LANES = 128


def _fused_body(xo_ref, xe_ref, xx_ref,
                w1a_ref, w1b_ref, w1c_ref, w1d_ref, w1e_ref, w1f_ref,
                b1_ref, w2_ref, b2_ref, w3_ref, b3_ref,
                out_ref,
                os_sum, os_max, es_sum, es_max, xs_sum, xs_max,
                *, num_chunks, bloc, inv_o, inv_e, inv_x):
    k = pl.program_id(1)

    @pl.when(k == 0)
    def _init():
        os_sum[...] = jnp.zeros_like(os_sum)
        es_sum[...] = jnp.zeros_like(es_sum)
        xs_sum[...] = jnp.zeros_like(xs_sum)
        os_max[...] = jnp.full_like(os_max, -jnp.inf)
        es_max[...] = jnp.full_like(es_max, -jnp.inf)
        xs_max[...] = jnp.full_like(xs_max, -jnp.inf)

    def _acc(ref, s_sc, m_sc):
        x = ref[...]
        r, c = x.shape
        x3 = x.reshape(r, c // _LANES, _LANES)
        s_sc[...] += jnp.sum(x3, axis=1)
        m_sc[...] = jnp.maximum(m_sc[...], jnp.max(x3, axis=1))

    _acc(xo_ref, os_sum, os_max)
    _acc(xe_ref, es_sum, es_max)
    _acc(xx_ref, xs_sum, xs_max)

    @pl.when(k == num_chunks - 1)
    def _finalize():
        def _feat(s_sc, m_sc, inv):
            r = s_sc.shape[0]
            a = jnp.sum(s_sc[...], axis=-1, keepdims=True) * inv
            m = jnp.max(m_sc[...], axis=-1, keepdims=True)
            return a.reshape(bloc, r // bloc), m.reshape(bloc, r // bloc)

        oa, om = _feat(os_sum, os_max, inv_o)
        ea, em = _feat(es_sum, es_max, inv_e)
        xa, xm = _feat(xs_sum, xs_max, inv_x)

        h = jnp.dot(xa, w1a_ref[...], preferred_element_type=jnp.float32)
        h = h + jnp.dot(xm, w1b_ref[...], preferred_element_type=jnp.float32)
        h = h + jnp.dot(ea, w1c_ref[...], preferred_element_type=jnp.float32)
        h = h + jnp.dot(em, w1d_ref[...], preferred_element_type=jnp.float32)
        h = h + jnp.dot(oa, w1e_ref[...], preferred_element_type=jnp.float32)
        h = h + jnp.dot(om, w1f_ref[...], preferred_element_type=jnp.float32)
        h = h + b1_ref[...]
        h = jnp.dot(h, w2_ref[...], preferred_element_type=jnp.float32) + b2_ref[...]
        y = jnp.dot(h, w3_ref[...], preferred_element_type=jnp.float32) + b3_ref[...]
        out_ref[...] = y.reshape(1, bloc, y.shape[-1])


def _pick_num_chunks(row_block, spatials, target_bytes):
    so = max(spatials)
    want = max(1, (row_block * so * 4 + target_bytes - 1) // target_bytes)
    legal = [n for n in range(1, 256)
             if all(s % n == 0 and (s // n) % _LANES == 0 for s in spatials)]
    at_least = [n for n in legal if n >= want]
    return min(at_least) if at_least else max(legal)


def kernel(x4_1, encoder_output, out_feature, w1, b1, w2, b2, w3, b3):
    B = int(x4_1.shape[0])
    cores = 2
    assert B % cores == 0
    bloc = B // cores

    def _flatten(x):
        c = int(x.shape[1])
        s = 1
        for d in x.shape[2:]:
            s *= int(d)
        return x.reshape(B * c, s), c, s

    xo, c_o, s_o = _flatten(out_feature)
    xe, c_e, s_e = _flatten(encoder_output)
    xx, c_x, s_x = _flatten(x4_1)

    ro, re, rx = B * c_o // cores, B * c_e // cores, B * c_x // cores
    nc = _pick_num_chunks(ro, (s_o, s_e, s_x), target_bytes=16 * _MIB)
    oc, ec, xc = s_o // nc, s_e // nc, s_x // nc

    widths = (c_x, c_x, c_e, c_e, c_o, c_o)
    offs = [0]
    for w in widths:
        offs.append(offs[-1] + w)
    assert offs[-1] == w1.shape[0]
    w1_parts = [w1[offs[i]:offs[i + 1], :] for i in range(6)]

    def _const_spec(a):
        return pl.BlockSpec(a.shape, lambda i, k: (0,) * a.ndim)

    weights = (*w1_parts, b1, w2, b2, w3, b3)
    n_out = int(w3.shape[1])

    body = functools.partial(
        _fused_body, num_chunks=nc, bloc=bloc,
        inv_o=1.0 / s_o, inv_e=1.0 / s_e, inv_x=1.0 / s_x)

    out = pl.pallas_call(
        body,
        out_shape=jax.ShapeDtypeStruct((cores, bloc, n_out), jnp.float32),
        grid=(cores, nc),
        in_specs=[
            pl.BlockSpec((ro, oc), lambda i, k: (i, k)),
            pl.BlockSpec((re, ec), lambda i, k: (i, k)),
            pl.BlockSpec((rx, xc), lambda i, k: (i, k)),
            *[_const_spec(a) for a in weights],
        ],
        out_specs=pl.BlockSpec((1, bloc, n_out), lambda i, k: (i, 0, 0)),
        scratch_shapes=[
            pltpu.VMEM((ro, _LANES), jnp.float32),
            pltpu.VMEM((ro, _LANES), jnp.float32),
            pltpu.VMEM((re, _LANES), jnp.float32),
            pltpu.VMEM((re, _LANES), jnp.float32),
            pltpu.VMEM((rx, _LANES), jnp.float32),
            pltpu.VMEM((rx, _LANES), jnp.float32),
        ],
        compiler_params=pltpu.CompilerParams(
            dimension_semantics=("parallel", "arbitrary"),
            vmem_limit_bytes=56 * _MIB,
        ),
    )(xo, xe, xx, *weights)
    return out.reshape(B, n_out)


# speedup vs baseline: 1.0047x; 1.0047x over previous
import functools

import jax
import jax.numpy as jnp
from jax.experimental import pallas as pl
from jax.experimental.pallas import tpu as pltpu

_MIB = 1024 * 1024
_LANES = 128


def _fused_body(xo_ref, xe_ref, xx_ref,
                w1_ref, b1_ref, w2_ref, b2_ref, w3_ref, b3_ref,
                out_ref,
                os_sum, os_max, es_sum, es_max, xs_sum, xs_max,
                *, num_chunks, bloc, widths, inv_o, inv_e, inv_x):
    k = pl.program_id(1)

    @pl.when(k == 0)
    def _init():
        os_sum[...] = jnp.zeros_like(os_sum)
        os_max[...] = jnp.full_like(os_max, -jnp.inf)

    xo = xo_ref[...]
    ro, oc = xo.shape
    xo3 = xo.reshape(ro, oc // _LANES, _LANES)
    os_sum[...] += jnp.sum(xo3, axis=1)
    os_max[...] = jnp.maximum(os_max[...], jnp.max(xo3, axis=1))

    def _rows(ref, s_sc, m_sc):
        x = ref[...]
        s_sc[k] = jnp.sum(x, axis=-1, keepdims=True)
        m_sc[k] = jnp.max(x, axis=-1, keepdims=True)

    _rows(xe_ref, es_sum, es_max)
    _rows(xx_ref, xs_sum, xs_max)

    @pl.when(k == num_chunks - 1)
    def _finalize():
        c_x, c_e, c_o = widths

        def _small(s_sc, m_sc, inv, c):
            a = s_sc[...].reshape(bloc, c) * inv
            m = m_sc[...].reshape(bloc, c)
            return a, m

        ea, em = _small(es_sum, es_max, inv_e, c_e)
        xa, xm = _small(xs_sum, xs_max, inv_x, c_x)
        r = os_sum.shape[0]
        oa = (jnp.sum(os_sum[...], axis=-1, keepdims=True) * inv_o).reshape(bloc, c_o)
        om = jnp.max(os_max[...], axis=-1, keepdims=True).reshape(bloc, c_o)

        offs = [0, c_x, 2 * c_x, 2 * c_x + c_e, 2 * c_x + 2 * c_e,
                2 * c_x + 2 * c_e + c_o, 2 * c_x + 2 * c_e + 2 * c_o]
        feats = (xa, xm, ea, em, oa, om)
        h = b1_ref[...]
        for f, lo, hi in zip(feats, offs[:-1], offs[1:]):
            h = h + jnp.dot(f, w1_ref[lo:hi, :],
                            preferred_element_type=jnp.float32)
        h = jnp.dot(h, w2_ref[...], preferred_element_type=jnp.float32) + b2_ref[...]
        y = jnp.dot(h, w3_ref[...], preferred_element_type=jnp.float32) + b3_ref[...]
        out_ref[...] = y.reshape(1, bloc, y.shape[-1])


def _pick_num_chunks(row_block, s_big, small_rows, target_bytes):
    want = max(1, (row_block * s_big * 4 + target_bytes - 1) // target_bytes)
    legal = [n for n in range(1, 257)
             if s_big % n == 0 and (s_big // n) % _LANES == 0
             and all(r % n == 0 and (r // n) % 8 == 0 for r in small_rows)]
    at_least = [n for n in legal if n >= want]
    return min(at_least) if at_least else max(legal)


def kernel(x4_1, encoder_output, out_feature, w1, b1, w2, b2, w3, b3):
    B = int(x4_1.shape[0])
    cores = 2
    assert B % cores == 0
    bloc = B // cores

    def _flatten(x):
        c = int(x.shape[1])
        s = 1
        for d in x.shape[2:]:
            s *= int(d)
        return x.reshape(B * c, s), c, s

    xo, c_o, s_o = _flatten(out_feature)
    xe, c_e, s_e = _flatten(encoder_output)
    xx, c_x, s_x = _flatten(x4_1)

    ro, re, rx = B * c_o // cores, B * c_e // cores, B * c_x // cores
    nc = _pick_num_chunks(ro, s_o, (re, rx), target_bytes=16 * _MIB)
    oc = s_o // nc
    re_b, rx_b = re // nc, rx // nc

    n_out = int(w3.shape[1])
    weights = (w1, b1, w2, b2, w3, b3)

    def _const_spec(a):
        return pl.BlockSpec(a.shape, lambda i, k: (0,) * a.ndim)

    body = functools.partial(
        _fused_body, num_chunks=nc, bloc=bloc, widths=(c_x, c_e, c_o),
        inv_o=1.0 / s_o, inv_e=1.0 / s_e, inv_x=1.0 / s_x)

    out = pl.pallas_call(
        body,
        out_shape=jax.ShapeDtypeStruct((cores, bloc, n_out), jnp.float32),
        grid=(cores, nc),
        in_specs=[
            pl.BlockSpec((ro, oc), lambda i, k: (i, k)),
            pl.BlockSpec((re_b, s_e), lambda i, k, _n=nc: (i * _n + k, 0)),
            pl.BlockSpec((rx_b, s_x), lambda i, k, _n=nc: (i * _n + k, 0)),
            *[_const_spec(a) for a in weights],
        ],
        out_specs=pl.BlockSpec((1, bloc, n_out), lambda i, k: (i, 0, 0)),
        scratch_shapes=[
            pltpu.VMEM((ro, _LANES), jnp.float32),
            pltpu.VMEM((ro, _LANES), jnp.float32),
            pltpu.VMEM((nc, re_b, 1), jnp.float32),
            pltpu.VMEM((nc, re_b, 1), jnp.float32),
            pltpu.VMEM((nc, rx_b, 1), jnp.float32),
            pltpu.VMEM((nc, rx_b, 1), jnp.float32),
        ],
        compiler_params=pltpu.CompilerParams(
            dimension_semantics=("parallel", "arbitrary"),
            vmem_limit_bytes=56 * _MIB,
        ),
    )(xo, xe, xx, *weights)
    return out.reshape(B, n_out)


# speedup vs baseline: 47.5958x; 47.3730x over previous
import functools

import jax
import jax.numpy as jnp
from jax.experimental import pallas as pl
from jax.experimental.pallas import tpu as pltpu

_MIB = 1024 * 1024
_LANES = 128


def _fused_body(xo_ref, xe_ref, xx_ref,
                w1_ref, b1_ref, w2_ref, b2_ref, w3_ref, b3_ref,
                out_ref,
                os_sum, os_max, es_sum, es_max, xs_sum, xs_max,
                *, nco, ns, bloc, widths, inv_o, inv_e, inv_x):
    k = pl.program_id(1)

    @pl.when(k < nco)
    def _big():
        x = xo_ref[...]
        xr = x.reshape(x.shape[1], x.shape[2])
        s = jnp.sum(xr, axis=0, keepdims=True)
        m = jnp.max(xr, axis=0, keepdims=True)
        os_sum[k] = jnp.sum(s, axis=-1, keepdims=True)
        os_max[k] = jnp.max(m, axis=-1, keepdims=True)

    @pl.when(k >= nco)
    def _small():
        kk = k - nco

        def _rows(ref, s_sc, m_sc):
            x = ref[...]
            s_sc[kk] = jnp.sum(x, axis=-1, keepdims=True)
            m_sc[kk] = jnp.max(x, axis=-1, keepdims=True)

        _rows(xe_ref, es_sum, es_max)
        _rows(xx_ref, xs_sum, xs_max)

    @pl.when(k == nco + ns - 1)
    def _finalize():
        c_x, c_e, c_o = widths
        oa = os_sum[...].reshape(bloc, c_o) * inv_o
        om = os_max[...].reshape(bloc, c_o)
        ea = es_sum[...].reshape(bloc, c_e) * inv_e
        em = es_max[...].reshape(bloc, c_e)
        xa = xs_sum[...].reshape(bloc, c_x) * inv_x
        xm = xs_max[...].reshape(bloc, c_x)

        offs = [0, c_x, 2 * c_x, 2 * c_x + c_e, 2 * c_x + 2 * c_e,
                2 * c_x + 2 * c_e + c_o, 2 * c_x + 2 * c_e + 2 * c_o]
        feats = (xa, xm, ea, em, oa, om)
        h = b1_ref[...]
        for f, lo, hi in zip(feats, offs[:-1], offs[1:]):
            h = h + jnp.dot(f, w1_ref[lo:hi, :],
                            preferred_element_type=jnp.float32)
        h = jnp.dot(h, w2_ref[...], preferred_element_type=jnp.float32) + b2_ref[...]
        y = jnp.dot(h, w3_ref[...], preferred_element_type=jnp.float32) + b3_ref[...]
        out_ref[...] = y.reshape(1, bloc, y.shape[-1])


def _pick_ns(small_rows, limit=16):
    for n in range(limit, 0, -1):
        if all(r % n == 0 and (r // n) % 8 == 0 for r in small_rows):
            return n
    return 1


def kernel(x4_1, encoder_output, out_feature, w1, b1, w2, b2, w3, b3):
    B = int(x4_1.shape[0])
    cores = 2
    assert B % cores == 0
    bloc = B // cores

    def _flatten(x):
        c = int(x.shape[1])
        s = 1
        for d in x.shape[2:]:
            s *= int(d)
        return x.reshape(B * c, s), c, s

    xo2, c_o, s_o = _flatten(out_feature)
    xe, c_e, s_e = _flatten(encoder_output)
    xx, c_x, s_x = _flatten(x4_1)

    assert s_o % _LANES == 0
    xo = xo2.reshape(B * c_o, s_o // _LANES, _LANES)

    ro, re, rx = B * c_o // cores, B * c_e // cores, B * c_x // cores
    nco = ro
    ns = _pick_ns((re, rx), limit=8)
    re_b, rx_b = re // ns, rx // ns

    n_out = int(w3.shape[1])
    weights = (w1, b1, w2, b2, w3, b3)

    def _const_spec(a):
        return pl.BlockSpec(a.shape, lambda i, k: (0,) * a.ndim)

    body = functools.partial(
        _fused_body, nco=nco, ns=ns, bloc=bloc, widths=(c_x, c_e, c_o),
        inv_o=1.0 / s_o, inv_e=1.0 / s_e, inv_x=1.0 / s_x)

    out = pl.pallas_call(
        body,
        out_shape=jax.ShapeDtypeStruct((cores, bloc, n_out), jnp.float32),
        grid=(cores, nco + ns),
        in_specs=[
            pl.BlockSpec(
                (1, s_o // _LANES, _LANES),
                lambda i, k, _n=nco: (i * _n + jnp.minimum(k, _n - 1), 0, 0)),
            pl.BlockSpec(
                (re_b, s_e),
                lambda i, k, _n=nco, _s=ns: (
                    i * _s + jnp.clip(k - _n, 0, _s - 1), 0)),
            pl.BlockSpec(
                (rx_b, s_x),
                lambda i, k, _n=nco, _s=ns: (
                    i * _s + jnp.clip(k - _n, 0, _s - 1), 0)),
            *[_const_spec(a) for a in weights],
        ],
        out_specs=pl.BlockSpec((1, bloc, n_out), lambda i, k: (i, 0, 0)),
        scratch_shapes=[
            pltpu.VMEM((nco, 1, 1), jnp.float32),
            pltpu.VMEM((nco, 1, 1), jnp.float32),
            pltpu.VMEM((ns, re_b, 1), jnp.float32),
            pltpu.VMEM((ns, re_b, 1), jnp.float32),
            pltpu.VMEM((ns, rx_b, 1), jnp.float32),
            pltpu.VMEM((ns, rx_b, 1), jnp.float32),
        ],
        compiler_params=pltpu.CompilerParams(
            dimension_semantics=("parallel", "arbitrary"),
            vmem_limit_bytes=56 * _MIB,
        ),
    )(xo, xe, xx, *weights)
    return out.reshape(B, n_out)


# speedup vs baseline: 48.9615x; 1.0287x over previous
import functools

import jax
import jax.numpy as jnp
from jax.experimental import pallas as pl
from jax.experimental.pallas import tpu as pltpu

_MIB = 1024 * 1024
_LANES = 128


def _fused_body(xo_ref, xe_ref, xx_ref,
                w1_ref, b1_ref, w2_ref, b2_ref, w3_ref, b3_ref,
                out_ref,
                os_sum, os_max, es_sum, es_max, xs_sum, xs_max,
                *, nco, ns, bloc, widths, inv_o, inv_e, inv_x):
    k = pl.program_id(1)

    x = xo_ref[...]
    xr = x.reshape(x.shape[1], x.shape[2])
    s = jnp.sum(xr, axis=0, keepdims=True)
    m = jnp.max(xr, axis=0, keepdims=True)
    os_sum[k] = jnp.sum(s, axis=-1, keepdims=True)
    os_max[k] = jnp.max(m, axis=-1, keepdims=True)

    @pl.when(k < ns)
    def _small():
        def _rows(ref, s_sc, m_sc):
            xs = ref[...]
            s_sc[k] = jnp.sum(xs, axis=-1, keepdims=True)
            m_sc[k] = jnp.max(xs, axis=-1, keepdims=True)

        _rows(xe_ref, es_sum, es_max)
        _rows(xx_ref, xs_sum, xs_max)

    @pl.when(k == nco - 1)
    def _finalize():
        c_x, c_e, c_o = widths
        oa = os_sum[...].reshape(bloc, c_o) * inv_o
        om = os_max[...].reshape(bloc, c_o)
        ea = es_sum[...].reshape(bloc, c_e) * inv_e
        em = es_max[...].reshape(bloc, c_e)
        xa = xs_sum[...].reshape(bloc, c_x) * inv_x
        xm = xs_max[...].reshape(bloc, c_x)

        offs = [0, c_x, 2 * c_x, 2 * c_x + c_e, 2 * c_x + 2 * c_e,
                2 * c_x + 2 * c_e + c_o, 2 * c_x + 2 * c_e + 2 * c_o]
        feats = (xa, xm, ea, em, oa, om)
        h = b1_ref[...]
        for f, lo, hi in zip(feats, offs[:-1], offs[1:]):
            h = h + jnp.dot(f, w1_ref[lo:hi, :],
                            preferred_element_type=jnp.float32)
        h = jnp.dot(h, w2_ref[...], preferred_element_type=jnp.float32) + b2_ref[...]
        y = jnp.dot(h, w3_ref[...], preferred_element_type=jnp.float32) + b3_ref[...]
        out_ref[...] = y.reshape(1, bloc, y.shape[-1])


def _pick_ns(small_rows, limit=16):
    for n in range(limit, 0, -1):
        if all(r % n == 0 and (r // n) % 8 == 0 for r in small_rows):
            return n
    return 1


def kernel(x4_1, encoder_output, out_feature, w1, b1, w2, b2, w3, b3):
    B = int(x4_1.shape[0])
    cores = 2
    assert B % cores == 0
    bloc = B // cores

    def _flatten(x):
        c = int(x.shape[1])
        s = 1
        for d in x.shape[2:]:
            s *= int(d)
        return x.reshape(B * c, s), c, s

    xo2, c_o, s_o = _flatten(out_feature)
    xe, c_e, s_e = _flatten(encoder_output)
    xx, c_x, s_x = _flatten(x4_1)

    assert s_o % _LANES == 0
    xo = xo2.reshape(B * c_o, s_o // _LANES, _LANES)

    ro, re, rx = B * c_o // cores, B * c_e // cores, B * c_x // cores
    nco = ro
    ns = _pick_ns((re, rx), limit=8)
    assert ns < nco
    re_b, rx_b = re // ns, rx // ns

    n_out = int(w3.shape[1])
    weights = (w1, b1, w2, b2, w3, b3)

    def _const_spec(a):
        return pl.BlockSpec(a.shape, lambda i, k: (0,) * a.ndim)

    body = functools.partial(
        _fused_body, nco=nco, ns=ns, bloc=bloc, widths=(c_x, c_e, c_o),
        inv_o=1.0 / s_o, inv_e=1.0 / s_e, inv_x=1.0 / s_x)

    out = pl.pallas_call(
        body,
        out_shape=jax.ShapeDtypeStruct((cores, bloc, n_out), jnp.float32),
        grid=(cores, nco),
        in_specs=[
            pl.BlockSpec(
                (1, s_o // _LANES, _LANES),
                lambda i, k, _n=nco: (i * _n + k, 0, 0)),
            pl.BlockSpec(
                (re_b, s_e),
                lambda i, k, _s=ns: (i * _s + jnp.minimum(k, _s - 1), 0)),
            pl.BlockSpec(
                (rx_b, s_x),
                lambda i, k, _s=ns: (i * _s + jnp.minimum(k, _s - 1), 0)),
            *[_const_spec(a) for a in weights],
        ],
        out_specs=pl.BlockSpec((1, bloc, n_out), lambda i, k: (i, 0, 0)),
        scratch_shapes=[
            pltpu.VMEM((nco, 1, 1), jnp.float32),
            pltpu.VMEM((nco, 1, 1), jnp.float32),
            pltpu.VMEM((ns, re_b, 1), jnp.float32),
            pltpu.VMEM((ns, re_b, 1), jnp.float32),
            pltpu.VMEM((ns, rx_b, 1), jnp.float32),
            pltpu.VMEM((ns, rx_b, 1), jnp.float32),
        ],
        compiler_params=pltpu.CompilerParams(
            dimension_semantics=("parallel", "arbitrary"),
            vmem_limit_bytes=56 * _MIB,
        ),
    )(xo, xe, xx, *weights)
    return out.reshape(B, n_out)


# speedup vs baseline: 53.1930x; 1.0864x over previous
import functools

import jax
import jax.numpy as jnp
from jax.experimental import pallas as pl
from jax.experimental.pallas import tpu as pltpu

_MIB = 1024 * 1024
_LANES = 128


def _fused_body(xo_ref, xe_ref, xx_ref,
                w1_ref, b1_ref, w2_ref, b2_ref, w3_ref, b3_ref,
                out_ref,
                os_sum, os_max, es_sum, es_max, xs_sum, xs_max,
                *, nco, ns, bloc, widths, inv_o, inv_e, inv_x):
    k = pl.program_id(1)

    x = xo_ref[...]
    s = jnp.sum(x, axis=1)
    m = jnp.max(x, axis=1)
    os_sum[k] = jnp.sum(s, axis=-1, keepdims=True)
    os_max[k] = jnp.max(m, axis=-1, keepdims=True)

    @pl.when(k < ns)
    def _small():
        def _rows(ref, s_sc, m_sc):
            xs = ref[...]
            s_sc[k] = jnp.sum(xs, axis=-1, keepdims=True)
            m_sc[k] = jnp.max(xs, axis=-1, keepdims=True)

        _rows(xe_ref, es_sum, es_max)
        _rows(xx_ref, xs_sum, xs_max)

    @pl.when(k == nco - 1)
    def _finalize():
        c_x, c_e, c_o = widths
        oa = os_sum[...].reshape(bloc, c_o) * inv_o
        om = os_max[...].reshape(bloc, c_o)
        ea = es_sum[...].reshape(bloc, c_e) * inv_e
        em = es_max[...].reshape(bloc, c_e)
        xa = xs_sum[...].reshape(bloc, c_x) * inv_x
        xm = xs_max[...].reshape(bloc, c_x)

        offs = [0, c_x, 2 * c_x, 2 * c_x + c_e, 2 * c_x + 2 * c_e,
                2 * c_x + 2 * c_e + c_o, 2 * c_x + 2 * c_e + 2 * c_o]
        feats = (xa, xm, ea, em, oa, om)
        h = b1_ref[...]
        for f, lo, hi in zip(feats, offs[:-1], offs[1:]):
            h = h + jnp.dot(f, w1_ref[lo:hi, :],
                            preferred_element_type=jnp.float32)
        h = jnp.dot(h, w2_ref[...], preferred_element_type=jnp.float32) + b2_ref[...]
        y = jnp.dot(h, w3_ref[...], preferred_element_type=jnp.float32) + b3_ref[...]
        out_ref[...] = y.reshape(1, bloc, y.shape[-1])


def _pick_ns(small_rows, limit=16):
    for n in range(limit, 0, -1):
        if all(r % n == 0 and (r // n) % 8 == 0 for r in small_rows):
            return n
    return 1


def kernel(x4_1, encoder_output, out_feature, w1, b1, w2, b2, w3, b3):
    B = int(x4_1.shape[0])
    cores = 2
    assert B % cores == 0
    bloc = B // cores

    def _flatten(x):
        c = int(x.shape[1])
        s = 1
        for d in x.shape[2:]:
            s *= int(d)
        return x.reshape(B * c, s), c, s

    xo2, c_o, s_o = _flatten(out_feature)
    xe, c_e, s_e = _flatten(encoder_output)
    xx, c_x, s_x = _flatten(x4_1)

    assert s_o % _LANES == 0
    xo = xo2.reshape(B * c_o, s_o // _LANES, _LANES)

    ro, re, rx = B * c_o // cores, B * c_e // cores, B * c_x // cores
    rpb = 2
    assert ro % rpb == 0
    nco = ro // rpb
    ns = _pick_ns((re, rx), limit=8)
    assert ns < nco
    re_b, rx_b = re // ns, rx // ns

    n_out = int(w3.shape[1])
    weights = (w1, b1, w2, b2, w3, b3)

    def _const_spec(a):
        return pl.BlockSpec(a.shape, lambda i, k: (0,) * a.ndim)

    body = functools.partial(
        _fused_body, nco=nco, ns=ns, bloc=bloc, widths=(c_x, c_e, c_o),
        inv_o=1.0 / s_o, inv_e=1.0 / s_e, inv_x=1.0 / s_x)

    out = pl.pallas_call(
        body,
        out_shape=jax.ShapeDtypeStruct((cores, bloc, n_out), jnp.float32),
        grid=(cores, nco),
        in_specs=[
            pl.BlockSpec(
                (rpb, s_o // _LANES, _LANES),
                lambda i, k, _n=nco: (i * _n + k, 0, 0)),
            pl.BlockSpec(
                (re_b, s_e),
                lambda i, k, _s=ns: (i * _s + jnp.minimum(k, _s - 1), 0)),
            pl.BlockSpec(
                (rx_b, s_x),
                lambda i, k, _s=ns: (i * _s + jnp.minimum(k, _s - 1), 0)),
            *[_const_spec(a) for a in weights],
        ],
        out_specs=pl.BlockSpec((1, bloc, n_out), lambda i, k: (i, 0, 0)),
        scratch_shapes=[
            pltpu.VMEM((nco, rpb, 1), jnp.float32),
            pltpu.VMEM((nco, rpb, 1), jnp.float32),
            pltpu.VMEM((ns, re_b, 1), jnp.float32),
            pltpu.VMEM((ns, re_b, 1), jnp.float32),
            pltpu.VMEM((ns, rx_b, 1), jnp.float32),
            pltpu.VMEM((ns, rx_b, 1), jnp.float32),
        ],
        compiler_params=pltpu.CompilerParams(
            dimension_semantics=("parallel", "arbitrary"),
            vmem_limit_bytes=56 * _MIB,
        ),
    )(xo, xe, xx, *weights)
    return out.reshape(B, n_out)


# speedup vs baseline: 53.2069x; 1.0003x over previous
import functools

import jax
import jax.numpy as jnp
from jax.experimental import pallas as pl
from jax.experimental.pallas import tpu as pltpu

_MIB = 1024 * 1024
_LANES = 128


def _fused_body(xo_ref, xe_ref, xx_ref,
                w1_ref, b1_ref, w2_ref, b2_ref, w3_ref, b3_ref,
                out_ref,
                os_sum, os_max, es_sum, es_max, xs_sum, xs_max,
                *, nco, ns, bloc, widths, inv_o, inv_e, inv_x):
    k = pl.program_id(1)

    x = xo_ref[...]
    s = jnp.sum(x, axis=1)
    m = jnp.max(x, axis=1)
    os_sum[k] = jnp.sum(s, axis=-1, keepdims=True)
    os_max[k] = jnp.max(m, axis=-1, keepdims=True)

    @pl.when(k < ns)
    def _small():
        def _rows(ref, s_sc, m_sc):
            xs = ref[...]
            s_sc[k] = jnp.sum(xs, axis=-1, keepdims=True)
            m_sc[k] = jnp.max(xs, axis=-1, keepdims=True)

        _rows(xe_ref, es_sum, es_max)
        _rows(xx_ref, xs_sum, xs_max)

    @pl.when(k == nco - 1)
    def _finalize():
        c_x, c_e, c_o = widths
        oa = os_sum[...].reshape(bloc, c_o) * inv_o
        om = os_max[...].reshape(bloc, c_o)
        ea = es_sum[...].reshape(bloc, c_e) * inv_e
        em = es_max[...].reshape(bloc, c_e)
        xa = xs_sum[...].reshape(bloc, c_x) * inv_x
        xm = xs_max[...].reshape(bloc, c_x)

        offs = [0, c_x, 2 * c_x, 2 * c_x + c_e, 2 * c_x + 2 * c_e,
                2 * c_x + 2 * c_e + c_o, 2 * c_x + 2 * c_e + 2 * c_o]
        feats = (xa, xm, ea, em, oa, om)
        h = b1_ref[...]
        for f, lo, hi in zip(feats, offs[:-1], offs[1:]):
            h = h + jnp.dot(f, w1_ref[lo:hi, :],
                            preferred_element_type=jnp.float32)
        h = jnp.dot(h, w2_ref[...], preferred_element_type=jnp.float32) + b2_ref[...]
        y = jnp.dot(h, w3_ref[...], preferred_element_type=jnp.float32) + b3_ref[...]
        out_ref[...] = y.reshape(1, bloc, y.shape[-1])


def _pick_ns(small_rows, limit=16):
    for n in range(limit, 0, -1):
        if all(r % n == 0 and (r // n) % 8 == 0 for r in small_rows):
            return n
    return 1


def kernel(x4_1, encoder_output, out_feature, w1, b1, w2, b2, w3, b3):
    B = int(x4_1.shape[0])
    cores = 2
    assert B % cores == 0
    bloc = B // cores

    def _flatten(x):
        c = int(x.shape[1])
        s = 1
        for d in x.shape[2:]:
            s *= int(d)
        return x.reshape(B * c, s), c, s

    xo2, c_o, s_o = _flatten(out_feature)
    xe, c_e, s_e = _flatten(encoder_output)
    xx, c_x, s_x = _flatten(x4_1)

    assert s_o % _LANES == 0
    xo = xo2.reshape(B * c_o, s_o // _LANES, _LANES)

    ro, re, rx = B * c_o // cores, B * c_e // cores, B * c_x // cores
    rpb = 2
    assert ro % rpb == 0
    nco = ro // rpb
    ns = _pick_ns((re, rx), limit=max(1, nco - 1))
    assert ns < nco
    re_b, rx_b = re // ns, rx // ns

    n_out = int(w3.shape[1])
    weights = (w1, b1, w2, b2, w3, b3)

    def _const_spec(a):
        return pl.BlockSpec(a.shape, lambda i, k: (0,) * a.ndim)

    body = functools.partial(
        _fused_body, nco=nco, ns=ns, bloc=bloc, widths=(c_x, c_e, c_o),
        inv_o=1.0 / s_o, inv_e=1.0 / s_e, inv_x=1.0 / s_x)

    out = pl.pallas_call(
        body,
        out_shape=jax.ShapeDtypeStruct((cores, bloc, n_out), jnp.float32),
        grid=(cores, nco),
        in_specs=[
            pl.BlockSpec(
                (rpb, s_o // _LANES, _LANES),
                lambda i, k, _n=nco: (i * _n + k, 0, 0)),
            pl.BlockSpec(
                (re_b, s_e),
                lambda i, k, _s=ns: (i * _s + jnp.minimum(k, _s - 1), 0)),
            pl.BlockSpec(
                (rx_b, s_x),
                lambda i, k, _s=ns: (i * _s + jnp.minimum(k, _s - 1), 0)),
            *[_const_spec(a) for a in weights],
        ],
        out_specs=pl.BlockSpec((1, bloc, n_out), lambda i, k: (i, 0, 0)),
        scratch_shapes=[
            pltpu.VMEM((nco, rpb, 1), jnp.float32),
            pltpu.VMEM((nco, rpb, 1), jnp.float32),
            pltpu.VMEM((ns, re_b, 1), jnp.float32),
            pltpu.VMEM((ns, re_b, 1), jnp.float32),
            pltpu.VMEM((ns, rx_b, 1), jnp.float32),
            pltpu.VMEM((ns, rx_b, 1), jnp.float32),
        ],
        compiler_params=pltpu.CompilerParams(
            dimension_semantics=("parallel", "arbitrary"),
            vmem_limit_bytes=56 * _MIB,
        ),
    )(xo, xe, xx, *weights)
    return out.reshape(B, n_out)
